# Initial kernel scaffold; baseline (speedup 1.0000x reference)
import jax
import jax.numpy as jnp
from jax.experimental import pallas as pl


def kernel(x, edge_index, y, mask, W1l, b1l, W1r, W2l, b2l, W2r):
    raise NotImplementedError("write your pallas kernel here")


# speedup vs baseline: 3.2907x; 3.2907x over previous
import functools

import jax
import jax.numpy as jnp
from jax import lax
from jax.experimental import pallas as pl
from jax.experimental.pallas import tpu as pltpu

N = 10000
E = 160000
D = 128
H = 128
C = 64
K = 16
ETA = 0.5

_INTERPRET = False


def _elu(v):
    return jnp.where(v > 0, v, jnp.exp(v) - 1.0)


def _layer1_body(p0_ref, p1_ref, x_ref, wl_ref, bl_ref, wr_ref, h_ref, deg_ref):
    s = p0_ref[...] + p1_ref[...]
    agg = s[:, :D]
    deg = jnp.maximum(s[:, D:D + 1], 1.0)
    mean = agg / deg
    z = (
        lax.dot_general(mean, wl_ref[...], (((1,), (1,)), ((), ())),
                        preferred_element_type=jnp.float32)
        + bl_ref[...]
        + lax.dot_general(x_ref[...], wr_ref[...], (((1,), (1,)), ((), ())),
                          preferred_element_type=jnp.float32)
    )
    h_ref[...] = _elu(z)
    deg_ref[...] = deg


def _layer1(parts0, parts1, x, W1l, b1l, W1r):
    R = 1000
    grid = N // R
    return pl.pallas_call(
        _layer1_body,
        grid=(grid,),
        in_specs=[
            pl.BlockSpec((R, 144), lambda i: (i, 0)),
            pl.BlockSpec((R, 144), lambda i: (i, 0)),
            pl.BlockSpec((R, D), lambda i: (i, 0)),
            pl.BlockSpec((H, D), lambda i: (0, 0)),
            pl.BlockSpec((1, H), lambda i: (0, 0)),
            pl.BlockSpec((H, D), lambda i: (0, 0)),
        ],
        out_specs=[
            pl.BlockSpec((R, H), lambda i: (i, 0)),
            pl.BlockSpec((R, 1), lambda i: (i, 0)),
        ],
        out_shape=[
            jax.ShapeDtypeStruct((N, H), jnp.float32),
            jax.ShapeDtypeStruct((N, 1), jnp.float32),
        ],
        interpret=_INTERPRET,
    )(parts0, parts1, x, W1l, b1l.reshape(1, H), W1r)


def _layer2_body(p0_ref, p1_ref, deg_ref, h_ref, wl_ref, bl_ref, wr_ref,
                 y_ref, plc_ref, hn_ref, oh_ref):
    s = p0_ref[...] + p1_ref[...]
    mean = s / deg_ref[...]
    h = h_ref[...]
    z = (
        lax.dot_general(mean, wl_ref[...], (((1,), (1,)), ((), ())),
                        preferred_element_type=jnp.float32)
        + bl_ref[...]
        + lax.dot_general(h, wr_ref[...], (((1,), (1,)), ((), ())),
                          preferred_element_type=jnp.float32)
    )
    lc = _elu(z)
    m = jnp.max(lc, axis=1, keepdims=True)
    plc_ref[...] = lc - m - jnp.log(jnp.sum(jnp.exp(lc - m), axis=1,
                                            keepdims=True))
    nrm = jnp.sqrt(jnp.sum(h * h, axis=1, keepdims=True))
    hn_ref[...] = h / jnp.maximum(nrm, 1e-8)
    cls = lax.broadcasted_iota(jnp.int32, oh_ref.shape, 1)
    oh_ref[...] = jnp.where(y_ref[...] == cls, 1.0, 0.0)


def _layer2(parts0, parts1, deg, h, W2l, b2l, W2r, y2d):
    R = 1000
    grid = N // R
    return pl.pallas_call(
        _layer2_body,
        grid=(grid,),
        in_specs=[
            pl.BlockSpec((R, H), lambda i: (i, 0)),
            pl.BlockSpec((R, H), lambda i: (i, 0)),
            pl.BlockSpec((R, 1), lambda i: (i, 0)),
            pl.BlockSpec((R, H), lambda i: (i, 0)),
            pl.BlockSpec((C, H), lambda i: (0, 0)),
            pl.BlockSpec((1, C), lambda i: (0, 0)),
            pl.BlockSpec((C, H), lambda i: (0, 0)),
            pl.BlockSpec((R, 1), lambda i: (i, 0)),
        ],
        out_specs=[
            pl.BlockSpec((R, C), lambda i: (i, 0)),
            pl.BlockSpec((R, H), lambda i: (i, 0)),
            pl.BlockSpec((R, C), lambda i: (i, 0)),
        ],
        out_shape=[
            jax.ShapeDtypeStruct((N, C), jnp.float32),
            jax.ShapeDtypeStruct((N, H), jnp.float32),
            jax.ShapeDtypeStruct((N, C), jnp.float32),
        ],
        interpret=_INTERPRET,
    )(parts0, parts1, deg, h, W2l, b2l.reshape(1, C), W2r, y2d)


_RB = 200


def _fuse_body(hn_full_ref, hn_blk_ref, mask_ref, plc_ref, oh_ref, out_ref):
    s = lax.dot_general(hn_blk_ref[...], hn_full_ref[...],
                        (((1,), (1,)), ((), ())),
                        preferred_element_type=jnp.float32)
    s = s * mask_ref[...]

    def step(_, cur):
        m = jnp.max(cur, axis=1, keepdims=True)
        return jnp.where(cur >= m, -jnp.inf, cur)

    cur = lax.fori_loop(0, K - 1, step, s)
    thresh = jnp.max(cur, axis=1, keepdims=True)
    w = jnp.where(s >= thresh, jnp.exp(s), 0.0)
    fuse = lax.dot_general(w, oh_ref[...], (((1,), (0,)), ((), ())),
                           preferred_element_type=jnp.float32)
    m = jnp.max(fuse, axis=1, keepdims=True)
    p_sim = fuse - m - jnp.log(jnp.sum(jnp.exp(fuse - m), axis=1,
                                       keepdims=True))
    out_ref[...] = ETA * plc_ref[...] + (1.0 - ETA) * p_sim


def _fused_sim(hn, mask, p_lc, onehot):
    grid = N // _RB
    return pl.pallas_call(
        _fuse_body,
        grid=(grid,),
        in_specs=[
            pl.BlockSpec((N, H), lambda i: (0, 0)),
            pl.BlockSpec((_RB, H), lambda i: (i, 0)),
            pl.BlockSpec((_RB, N), lambda i: (i, 0)),
            pl.BlockSpec((_RB, C), lambda i: (i, 0)),
            pl.BlockSpec((N, C), lambda i: (0, 0)),
        ],
        out_specs=pl.BlockSpec((_RB, C), lambda i: (i, 0)),
        out_shape=jax.ShapeDtypeStruct((N, C), jnp.float32),
        interpret=_INTERPRET,
    )(hn, hn, mask, p_lc, onehot)


def _segment_sum_parts(feat, src, dst):
    Dw = feat.shape[1]
    half = E // 2
    p0 = jax.ops.segment_sum(jnp.take(feat, src[:half], axis=0), dst[:half],
                             num_segments=N)
    p1 = jax.ops.segment_sum(jnp.take(feat, src[half:], axis=0), dst[half:],
                             num_segments=N)
    return jnp.stack([p0, p1])


def kernel(x, edge_index, y, mask, W1l, b1l, W1r, W2l, b2l, W2r):
    src = edge_index[0]
    dst = edge_index[1]
    ones_col = jnp.concatenate(
        [jnp.ones((N, 1), jnp.float32), jnp.zeros((N, 15), jnp.float32)], axis=1)
    x_ext = jnp.concatenate([x, ones_col], axis=1)

    parts1 = _segment_sum_parts(x_ext, src, dst)
    h, deg = _layer1(parts1[0], parts1[1], x, W1l, b1l, W1r)

    parts2 = _segment_sum_parts(h, src, dst)
    p_lc, hn, onehot = _layer2(parts2[0], parts2[1], deg, h, W2l, b2l, W2r,
                               y.reshape(N, 1))

    final = _fused_sim(hn, mask, p_lc, onehot)
    return (final, h)


# speedup vs baseline: 4.2305x; 1.2856x over previous
import functools

import jax
import jax.numpy as jnp
from jax import lax
from jax.experimental import pallas as pl
from jax.experimental.pallas import tpu as pltpu
from jax.experimental.pallas import tpu_sc as plsc

N = 10000
E = 160000
D = 128
H = 128
C = 64
K = 16
ETA = 0.5

_INTERPRET = False


def _elu(v):
    return jnp.where(v > 0, v, jnp.exp(v) - 1.0)


def _layer1_body(p0_ref, p1_ref, d0_ref, d1_ref, x_ref, wl_ref, bl_ref,
                 wr_ref, h_ref):
    deg = jnp.maximum(d0_ref[...] + d1_ref[...], 1.0)
    mean = (p0_ref[...] + p1_ref[...]) / deg
    z = (
        lax.dot_general(mean, wl_ref[...], (((1,), (1,)), ((), ())),
                        preferred_element_type=jnp.float32)
        + bl_ref[...]
        + lax.dot_general(x_ref[...], wr_ref[...], (((1,), (1,)), ((), ())),
                          preferred_element_type=jnp.float32)
    )
    h_ref[...] = _elu(z)


def _layer1(parts0, parts1, deg0, deg1, x, W1l, b1l, W1r):
    R = 1000
    grid = N // R
    return pl.pallas_call(
        _layer1_body,
        grid=(grid,),
        in_specs=[
            pl.BlockSpec((R, D), lambda i: (i, 0)),
            pl.BlockSpec((R, D), lambda i: (i, 0)),
            pl.BlockSpec((R, 1), lambda i: (i, 0)),
            pl.BlockSpec((R, 1), lambda i: (i, 0)),
            pl.BlockSpec((R, D), lambda i: (i, 0)),
            pl.BlockSpec((H, D), lambda i: (0, 0)),
            pl.BlockSpec((1, H), lambda i: (0, 0)),
            pl.BlockSpec((H, D), lambda i: (0, 0)),
        ],
        out_specs=pl.BlockSpec((R, H), lambda i: (i, 0)),
        out_shape=jax.ShapeDtypeStruct((N, H), jnp.float32),
        interpret=_INTERPRET,
    )(parts0, parts1, deg0, deg1, x, W1l, b1l.reshape(1, H), W1r)


def _layer2_body(p0_ref, p1_ref, d0_ref, d1_ref, h_ref, wl_ref, bl_ref,
                 wr_ref, y_ref, plc_ref, hn_ref, oh_ref):
    deg = jnp.maximum(d0_ref[...] + d1_ref[...], 1.0)
    mean = (p0_ref[...] + p1_ref[...]) / deg
    h = h_ref[...]
    z = (
        lax.dot_general(mean, wl_ref[...], (((1,), (1,)), ((), ())),
                        preferred_element_type=jnp.float32)
        + bl_ref[...]
        + lax.dot_general(h, wr_ref[...], (((1,), (1,)), ((), ())),
                          preferred_element_type=jnp.float32)
    )
    lc = _elu(z)
    m = jnp.max(lc, axis=1, keepdims=True)
    plc_ref[...] = lc - m - jnp.log(jnp.sum(jnp.exp(lc - m), axis=1,
                                            keepdims=True))
    nrm = jnp.sqrt(jnp.sum(h * h, axis=1, keepdims=True))
    hn_ref[...] = h / jnp.maximum(nrm, 1e-8)
    cls = lax.broadcasted_iota(jnp.int32, oh_ref.shape, 1)
    oh_ref[...] = jnp.where(y_ref[...] == cls, 1.0, 0.0)


def _layer2(parts0, parts1, deg0, deg1, h, W2l, b2l, W2r, y2d):
    R = 1000
    grid = N // R
    return pl.pallas_call(
        _layer2_body,
        grid=(grid,),
        in_specs=[
            pl.BlockSpec((R, H), lambda i: (i, 0)),
            pl.BlockSpec((R, H), lambda i: (i, 0)),
            pl.BlockSpec((R, 1), lambda i: (i, 0)),
            pl.BlockSpec((R, 1), lambda i: (i, 0)),
            pl.BlockSpec((R, H), lambda i: (i, 0)),
            pl.BlockSpec((C, H), lambda i: (0, 0)),
            pl.BlockSpec((1, C), lambda i: (0, 0)),
            pl.BlockSpec((C, H), lambda i: (0, 0)),
            pl.BlockSpec((R, 1), lambda i: (i, 0)),
        ],
        out_specs=[
            pl.BlockSpec((R, C), lambda i: (i, 0)),
            pl.BlockSpec((R, H), lambda i: (i, 0)),
            pl.BlockSpec((R, C), lambda i: (i, 0)),
        ],
        out_shape=[
            jax.ShapeDtypeStruct((N, C), jnp.float32),
            jax.ShapeDtypeStruct((N, H), jnp.float32),
            jax.ShapeDtypeStruct((N, C), jnp.float32),
        ],
        interpret=_INTERPRET,
    )(parts0, parts1, deg0, deg1, h, W2l, b2l.reshape(1, C), W2r, y2d)


_RB = 200


def _fuse_body(hn_full_ref, hn_blk_ref, mask_ref, plc_ref, oh_ref, out_ref):
    s = lax.dot_general(hn_blk_ref[...], hn_full_ref[...],
                        (((1,), (1,)), ((), ())),
                        preferred_element_type=jnp.float32)
    s = s * mask_ref[...]

    def step(_, cur):
        m = jnp.max(cur, axis=1, keepdims=True)
        return jnp.where(cur >= m, -jnp.inf, cur)

    cur = lax.fori_loop(0, K - 1, step, s)
    thresh = jnp.max(cur, axis=1, keepdims=True)
    w = jnp.where(s >= thresh, jnp.exp(s), 0.0)
    fuse = lax.dot_general(w, oh_ref[...], (((1,), (0,)), ((), ())),
                           preferred_element_type=jnp.float32)
    m = jnp.max(fuse, axis=1, keepdims=True)
    p_sim = fuse - m - jnp.log(jnp.sum(jnp.exp(fuse - m), axis=1,
                                       keepdims=True))
    out_ref[...] = ETA * plc_ref[...] + (1.0 - ETA) * p_sim


def _fused_sim(hn, mask, p_lc, onehot):
    grid = N // _RB
    return pl.pallas_call(
        _fuse_body,
        grid=(grid,),
        in_specs=[
            pl.BlockSpec((N, H), lambda i: (0, 0)),
            pl.BlockSpec((_RB, H), lambda i: (i, 0)),
            pl.BlockSpec((_RB, N), lambda i: (i, 0)),
            pl.BlockSpec((_RB, C), lambda i: (i, 0)),
            pl.BlockSpec((N, C), lambda i: (0, 0)),
        ],
        out_specs=pl.BlockSpec((_RB, C), lambda i: (i, 0)),
        out_shape=jax.ShapeDtypeStruct((N, C), jnp.float32),
        interpret=_INTERPRET,
    )(hn, hn, mask, p_lc, onehot)


_NB = 1280
_BPW = _NB // 32
_NPAD = 10240
_RPS = _NPAD // 16


def _make_sc_segsum(with_deg):
    mesh = plsc.VectorSubcoreMesh(core_axis_name="c", subcore_axis_name="s")
    out_type = [jax.ShapeDtypeStruct((2, _NPAD, D), jnp.float32)]
    scratch = [
        pltpu.VMEM((_BPW, 128), jnp.int32),
        pltpu.VMEM((_BPW, 128), jnp.int32),
        pltpu.VMEM((128, D), jnp.float32),
        pltpu.VMEM((32, D), jnp.float32),
        pltpu.VMEM_SHARED((_NPAD, D), jnp.float32),
        pltpu.SemaphoreType.DMA,
    ]
    if with_deg:
        out_type.append(jax.ShapeDtypeStruct((2, _NPAD // 128, 128),
                                             jnp.float32))
        scratch += [
            pltpu.VMEM((1, 128), jnp.int32),
            pltpu.VMEM((1, 128), jnp.int32),
            pltpu.VMEM_SHARED((_NPAD // 128, 128), jnp.float32),
        ]

    @functools.partial(pl.kernel, out_type=out_type, mesh=mesh,
                       scratch_types=scratch)
    def segsum(*args):
        if with_deg:
            (feat, src2d, dst2d, eye, out, dout,
             sidx, didx, rows, zbuf, acc, sem, dmrow, ddrow, dacc) = args
        else:
            feat, src2d, dst2d, out, sidx, didx, rows, zbuf, acc, sem = args
        c = lax.axis_index("c")
        s = lax.axis_index("s")
        w = c * 16 + s

        def zrow(r, carry):
            for j in range(D // 16):
                zbuf[r, pl.ds(j * 16, 16)] = jnp.zeros((16,), jnp.float32)
            return carry

        lax.fori_loop(0, 32, zrow, 0)
        for i in range(20):
            pltpu.sync_copy(zbuf, acc.at[pl.ds(s * _RPS + i * 32, 32)])
        if with_deg:
            @pl.when(s == 0)
            def _():
                for i in range(_NPAD // 128 // 16):
                    pltpu.sync_copy(zbuf.at[pl.ds(0, 16)],
                                    dacc.at[pl.ds(i * 16, 16)])
        plsc.subcore_barrier()

        pltpu.sync_copy(src2d.at[pl.ds(w * _BPW, _BPW)], sidx)
        pltpu.sync_copy(dst2d.at[pl.ds(w * _BPW, _BPW)], didx)

        def batch(j, carry):
            pltpu.async_copy(feat.at[sidx.at[j]], rows, sem).wait()
            pltpu.sync_copy(rows, acc.at[didx.at[j]], add=True)
            if with_deg:
                for k in range(8):
                    dv = didx[j, pl.ds(k * 16, 16)]
                    dmrow[0, pl.ds(k * 16, 16)] = lax.rem(dv, 128)
                    ddrow[0, pl.ds(k * 16, 16)] = lax.div(dv, 128)
                pltpu.async_copy(eye.at[dmrow.at[0]], rows, sem).wait()
                pltpu.sync_copy(rows, dacc.at[ddrow.at[0]], add=True)
            return carry

        lax.fori_loop(0, _BPW, batch, 0)
        plsc.subcore_barrier()

        for i in range(20):
            r0 = s * _RPS + i * 32
            pltpu.sync_copy(acc.at[pl.ds(r0, 32)], zbuf)
            pltpu.sync_copy(zbuf, out.at[c].at[pl.ds(r0, 32)])
        if with_deg:
            @pl.when(s == 1)
            def _():
                pltpu.sync_copy(dacc, rows.at[pl.ds(0, _NPAD // 128)])
                pltpu.sync_copy(rows.at[pl.ds(0, _NPAD // 128)], dout.at[c])

    return segsum


_sc_segsum_deg = _make_sc_segsum(True)
_sc_segsum = _make_sc_segsum(False)


def kernel(x, edge_index, y, mask, W1l, b1l, W1r, W2l, b2l, W2r):
    npad = _NB * 128 - E
    src2d = jnp.concatenate(
        [edge_index[0], jnp.zeros((npad,), jnp.int32)]).reshape(_NB, 128)
    dst_flat = jnp.concatenate(
        [edge_index[1], jnp.full((npad,), N + 16, jnp.int32)])
    dst2d = dst_flat.reshape(_NB, 128)
    eye = jnp.eye(128, dtype=jnp.float32)

    parts1, degp = _sc_segsum_deg(x, src2d, dst2d, eye)
    deg0 = degp[0].reshape(_NPAD, 1)
    deg1 = degp[1].reshape(_NPAD, 1)
    h = _layer1(parts1[0], parts1[1], deg0, deg1, x, W1l, b1l, W1r)

    (parts2,) = _sc_segsum(h, src2d, dst2d)
    p_lc, hn, onehot = _layer2(parts2[0], parts2[1], deg0, deg1, h, W2l, b2l,
                               W2r, y.reshape(N, 1))

    final = _fused_sim(hn, mask, p_lc, onehot)
    return (final, h)


# speedup vs baseline: 8.2367x; 1.9470x over previous
import functools

import jax
import jax.numpy as jnp
from jax import lax
from jax.experimental import pallas as pl
from jax.experimental.pallas import tpu as pltpu
from jax.experimental.pallas import tpu_sc as plsc

N = 10000
E = 160000
D = 128
H = 128
C = 64
K = 16
ETA = 0.5

_INTERPRET = False


def _elu(v):
    return jnp.where(v > 0, v, jnp.exp(v) - 1.0)


def _layer1_body(p0_ref, p1_ref, d0_ref, d1_ref, x_ref, wl_ref, bl_ref,
                 wr_ref, h_ref):
    deg = jnp.maximum(d0_ref[...] + d1_ref[...], 1.0)
    mean = (p0_ref[...] + p1_ref[...]) / deg
    z = (
        lax.dot_general(mean, wl_ref[...], (((1,), (1,)), ((), ())),
                        preferred_element_type=jnp.float32)
        + bl_ref[...]
        + lax.dot_general(x_ref[...], wr_ref[...], (((1,), (1,)), ((), ())),
                          preferred_element_type=jnp.float32)
    )
    h_ref[...] = _elu(z)


def _layer1(parts0, parts1, deg0, deg1, x, W1l, b1l, W1r):
    R = 1000
    grid = N // R
    return pl.pallas_call(
        _layer1_body,
        grid=(grid,),
        in_specs=[
            pl.BlockSpec((R, D), lambda i: (i, 0)),
            pl.BlockSpec((R, D), lambda i: (i, 0)),
            pl.BlockSpec((R, 1), lambda i: (i, 0)),
            pl.BlockSpec((R, 1), lambda i: (i, 0)),
            pl.BlockSpec((R, D), lambda i: (i, 0)),
            pl.BlockSpec((H, D), lambda i: (0, 0)),
            pl.BlockSpec((1, H), lambda i: (0, 0)),
            pl.BlockSpec((H, D), lambda i: (0, 0)),
        ],
        out_specs=pl.BlockSpec((R, H), lambda i: (i, 0)),
        out_shape=jax.ShapeDtypeStruct((N, H), jnp.float32),
        interpret=_INTERPRET,
    )(parts0, parts1, deg0, deg1, x, W1l, b1l.reshape(1, H), W1r)


def _layer2_body(p0_ref, p1_ref, d0_ref, d1_ref, h_ref, wl_ref, bl_ref,
                 wr_ref, y_ref, plc_ref, hn_ref, oh_ref):
    deg = jnp.maximum(d0_ref[...] + d1_ref[...], 1.0)
    mean = (p0_ref[...] + p1_ref[...]) / deg
    h = h_ref[...]
    z = (
        lax.dot_general(mean, wl_ref[...], (((1,), (1,)), ((), ())),
                        preferred_element_type=jnp.float32)
        + bl_ref[...]
        + lax.dot_general(h, wr_ref[...], (((1,), (1,)), ((), ())),
                          preferred_element_type=jnp.float32)
    )
    lc = _elu(z)
    m = jnp.max(lc, axis=1, keepdims=True)
    plc_ref[...] = lc - m - jnp.log(jnp.sum(jnp.exp(lc - m), axis=1,
                                            keepdims=True))
    nrm = jnp.sqrt(jnp.sum(h * h, axis=1, keepdims=True))
    hn_ref[...] = h / jnp.maximum(nrm, 1e-8)
    cls = lax.broadcasted_iota(jnp.int32, oh_ref.shape, 1)
    oh_ref[...] = jnp.where(y_ref[...] == cls, 1.0, 0.0)


def _layer2(parts0, parts1, deg0, deg1, h, W2l, b2l, W2r, y2d):
    R = 1000
    grid = N // R
    return pl.pallas_call(
        _layer2_body,
        grid=(grid,),
        in_specs=[
            pl.BlockSpec((R, H), lambda i: (i, 0)),
            pl.BlockSpec((R, H), lambda i: (i, 0)),
            pl.BlockSpec((R, 1), lambda i: (i, 0)),
            pl.BlockSpec((R, 1), lambda i: (i, 0)),
            pl.BlockSpec((R, H), lambda i: (i, 0)),
            pl.BlockSpec((C, H), lambda i: (0, 0)),
            pl.BlockSpec((1, C), lambda i: (0, 0)),
            pl.BlockSpec((C, H), lambda i: (0, 0)),
            pl.BlockSpec((R, 1), lambda i: (i, 0)),
        ],
        out_specs=[
            pl.BlockSpec((R, C), lambda i: (i, 0)),
            pl.BlockSpec((R, H), lambda i: (i, 0)),
            pl.BlockSpec((R, C), lambda i: (i, 0)),
        ],
        out_shape=[
            jax.ShapeDtypeStruct((N, C), jnp.float32),
            jax.ShapeDtypeStruct((N, H), jnp.float32),
            jax.ShapeDtypeStruct((N, C), jnp.float32),
        ],
        interpret=_INTERPRET,
    )(parts0, parts1, deg0, deg1, h, W2l, b2l.reshape(1, C), W2r, y2d)


_RB = 200


_GW = 1280


def _fuse_body(hn_full_ref, hn_blk_ref, mask_ref, plc_ref, oh_ref, out_ref,
               t7_ref):
    hnb = hn_blk_ref[...]
    sm = []
    for g in range(8):
        w0 = _GW * g
        wd = _GW if g < 7 else N - 7 * _GW
        sg = lax.dot_general(hnb, hn_full_ref[w0:w0 + wd, :],
                             (((1,), (1,)), ((), ())),
                             preferred_element_type=jnp.float32)
        sm.append(sg * mask_ref[:, w0:w0 + wd])

    t7_ref[...] = jnp.full(t7_ref.shape, -jnp.inf, jnp.float32)
    t7_ref[:, :N - 7 * _GW] = sm[7]
    members = sm[:7] + [t7_ref[...]]

    t1 = functools.reduce(jnp.maximum, members)
    t2 = functools.reduce(
        jnp.maximum,
        [jnp.where(mg == t1, -jnp.inf, mg) for mg in members])

    def step(_, carry):
        c1, c2 = carry
        m = jnp.max(c1, axis=1, keepdims=True)
        win = c1 >= m
        return jnp.where(win, c2, c1), jnp.where(win, -jnp.inf, c2)

    t1f, _ = lax.fori_loop(0, K - 1, step, (t1, t2))
    thresh = jnp.max(t1f, axis=1, keepdims=True)

    fuse = jnp.zeros((out_ref.shape[0], C), jnp.float32)
    for g in range(8):
        w0 = _GW * g
        wd = _GW if g < 7 else N - 7 * _GW
        wg = jnp.where(sm[g] >= thresh, jnp.exp(sm[g]), 0.0)
        fuse = fuse + lax.dot_general(wg, oh_ref[w0:w0 + wd, :],
                                      (((1,), (0,)), ((), ())),
                                      preferred_element_type=jnp.float32)
    m = jnp.max(fuse, axis=1, keepdims=True)
    p_sim = fuse - m - jnp.log(jnp.sum(jnp.exp(fuse - m), axis=1,
                                       keepdims=True))
    out_ref[...] = ETA * plc_ref[...] + (1.0 - ETA) * p_sim


def _fused_sim(hn, mask, p_lc, onehot):
    grid = N // _RB
    return pl.pallas_call(
        _fuse_body,
        grid=(grid,),
        in_specs=[
            pl.BlockSpec((N, H), lambda i: (0, 0)),
            pl.BlockSpec((_RB, H), lambda i: (i, 0)),
            pl.BlockSpec((_RB, N), lambda i: (i, 0)),
            pl.BlockSpec((_RB, C), lambda i: (i, 0)),
            pl.BlockSpec((N, C), lambda i: (0, 0)),
        ],
        out_specs=pl.BlockSpec((_RB, C), lambda i: (i, 0)),
        out_shape=jax.ShapeDtypeStruct((N, C), jnp.float32),
        scratch_shapes=[pltpu.VMEM((_RB, _GW), jnp.float32)],
        interpret=_INTERPRET,
    )(hn, hn, mask, p_lc, onehot)


_NB = 1280
_BPW = _NB // 32
_NPAD = 10240
_RPS = _NPAD // 16


def _make_sc_segsum(with_deg):
    mesh = plsc.VectorSubcoreMesh(core_axis_name="c", subcore_axis_name="s")
    out_type = [jax.ShapeDtypeStruct((2, _NPAD, D), jnp.float32)]
    scratch = [
        pltpu.VMEM((_BPW, 128), jnp.int32),
        pltpu.VMEM((_BPW, 128), jnp.int32),
        pltpu.VMEM((128, D), jnp.float32),
        pltpu.VMEM((32, D), jnp.float32),
        pltpu.VMEM_SHARED((_NPAD, D), jnp.float32),
        pltpu.SemaphoreType.DMA,
    ]
    if with_deg:
        out_type.append(jax.ShapeDtypeStruct((2, _NPAD // 128, 128),
                                             jnp.float32))
        scratch += [
            pltpu.VMEM((1, 128), jnp.int32),
            pltpu.VMEM((1, 128), jnp.int32),
            pltpu.VMEM_SHARED((_NPAD // 128, 128), jnp.float32),
        ]

    @functools.partial(pl.kernel, out_type=out_type, mesh=mesh,
                       scratch_types=scratch)
    def segsum(*args):
        if with_deg:
            (feat, src2d, dst2d, eye, out, dout,
             sidx, didx, rows, zbuf, acc, sem, dmrow, ddrow, dacc) = args
        else:
            feat, src2d, dst2d, out, sidx, didx, rows, zbuf, acc, sem = args
        c = lax.axis_index("c")
        s = lax.axis_index("s")
        w = c * 16 + s

        def zrow(r, carry):
            for j in range(D // 16):
                zbuf[r, pl.ds(j * 16, 16)] = jnp.zeros((16,), jnp.float32)
            return carry

        lax.fori_loop(0, 32, zrow, 0)
        for i in range(20):
            pltpu.sync_copy(zbuf, acc.at[pl.ds(s * _RPS + i * 32, 32)])
        if with_deg:
            @pl.when(s == 0)
            def _():
                for i in range(_NPAD // 128 // 16):
                    pltpu.sync_copy(zbuf.at[pl.ds(0, 16)],
                                    dacc.at[pl.ds(i * 16, 16)])
        plsc.subcore_barrier()

        pltpu.sync_copy(src2d.at[pl.ds(w * _BPW, _BPW)], sidx)
        pltpu.sync_copy(dst2d.at[pl.ds(w * _BPW, _BPW)], didx)

        def batch(j, carry):
            pltpu.async_copy(feat.at[sidx.at[j]], rows, sem).wait()
            pltpu.sync_copy(rows, acc.at[didx.at[j]], add=True)
            if with_deg:
                for k in range(8):
                    dv = didx[j, pl.ds(k * 16, 16)]
                    dmrow[0, pl.ds(k * 16, 16)] = lax.rem(dv, 128)
                    ddrow[0, pl.ds(k * 16, 16)] = lax.div(dv, 128)
                pltpu.async_copy(eye.at[dmrow.at[0]], rows, sem).wait()
                pltpu.sync_copy(rows, dacc.at[ddrow.at[0]], add=True)
            return carry

        lax.fori_loop(0, _BPW, batch, 0)
        plsc.subcore_barrier()

        for i in range(20):
            r0 = s * _RPS + i * 32
            pltpu.sync_copy(acc.at[pl.ds(r0, 32)], zbuf)
            pltpu.sync_copy(zbuf, out.at[c].at[pl.ds(r0, 32)])
        if with_deg:
            @pl.when(s == 1)
            def _():
                pltpu.sync_copy(dacc, rows.at[pl.ds(0, _NPAD // 128)])
                pltpu.sync_copy(rows.at[pl.ds(0, _NPAD // 128)], dout.at[c])

    return segsum


_sc_segsum_deg = _make_sc_segsum(True)
_sc_segsum = _make_sc_segsum(False)


def kernel(x, edge_index, y, mask, W1l, b1l, W1r, W2l, b2l, W2r):
    npad = _NB * 128 - E
    src2d = jnp.concatenate(
        [edge_index[0], jnp.zeros((npad,), jnp.int32)]).reshape(_NB, 128)
    dst_flat = jnp.concatenate(
        [edge_index[1], jnp.full((npad,), N + 16, jnp.int32)])
    dst2d = dst_flat.reshape(_NB, 128)
    eye = jnp.eye(128, dtype=jnp.float32)

    parts1, degp = _sc_segsum_deg(x, src2d, dst2d, eye)
    deg0 = degp[0].reshape(_NPAD, 1)
    deg1 = degp[1].reshape(_NPAD, 1)
    h = _layer1(parts1[0], parts1[1], deg0, deg1, x, W1l, b1l, W1r)

    (parts2,) = _sc_segsum(h, src2d, dst2d)
    p_lc, hn, onehot = _layer2(parts2[0], parts2[1], deg0, deg1, h, W2l, b2l,
                               W2r, y.reshape(N, 1))

    final = _fused_sim(hn, mask, p_lc, onehot)
    return (final, h)


# speedup vs baseline: 11.6629x; 1.4160x over previous
import functools

import jax
import jax.numpy as jnp
from jax import lax
from jax.experimental import pallas as pl
from jax.experimental.pallas import tpu as pltpu
from jax.experimental.pallas import tpu_sc as plsc

N = 10000
E = 160000
D = 128
H = 128
C = 64
K = 16
ETA = 0.5

_INTERPRET = False


def _elu(v):
    return jnp.where(v > 0, v, jnp.exp(v) - 1.0)


def _layer1_body(p0_ref, p1_ref, d0_ref, d1_ref, x_ref, wl_ref, bl_ref,
                 wr_ref, y_ref, h_ref, hn_ref, oh_ref):
    deg = jnp.maximum(d0_ref[...] + d1_ref[...], 1.0)
    mean = (p0_ref[...] + p1_ref[...]) / deg
    z = (
        lax.dot_general(mean, wl_ref[...], (((1,), (1,)), ((), ())),
                        preferred_element_type=jnp.float32)
        + bl_ref[...]
        + lax.dot_general(x_ref[...], wr_ref[...], (((1,), (1,)), ((), ())),
                          preferred_element_type=jnp.float32)
    )
    h = _elu(z)
    h_ref[...] = h
    nrm = jnp.sqrt(jnp.sum(h * h, axis=1, keepdims=True))
    hn_ref[...] = h / jnp.maximum(nrm, 1e-8)
    cls = lax.broadcasted_iota(jnp.int32, oh_ref.shape, 1)
    oh_ref[...] = jnp.where(y_ref[...] == cls, 1.0, 0.0)


def _layer1(parts0, parts1, deg0, deg1, x, W1l, b1l, W1r, y2d):
    R = 1000
    grid = N // R
    return pl.pallas_call(
        _layer1_body,
        grid=(grid,),
        in_specs=[
            pl.BlockSpec((R, D), lambda i: (i, 0)),
            pl.BlockSpec((R, D), lambda i: (i, 0)),
            pl.BlockSpec((R, 1), lambda i: (i, 0)),
            pl.BlockSpec((R, 1), lambda i: (i, 0)),
            pl.BlockSpec((R, D), lambda i: (i, 0)),
            pl.BlockSpec((H, D), lambda i: (0, 0)),
            pl.BlockSpec((1, H), lambda i: (0, 0)),
            pl.BlockSpec((H, D), lambda i: (0, 0)),
            pl.BlockSpec((R, 1), lambda i: (i, 0)),
        ],
        out_specs=[
            pl.BlockSpec((R, H), lambda i: (i, 0)),
            pl.BlockSpec((R, H), lambda i: (i, 0)),
            pl.BlockSpec((R, C), lambda i: (i, 0)),
        ],
        out_shape=[
            jax.ShapeDtypeStruct((N, H), jnp.float32),
            jax.ShapeDtypeStruct((N, H), jnp.float32),
            jax.ShapeDtypeStruct((N, C), jnp.float32),
        ],
        interpret=_INTERPRET,
    )(parts0, parts1, deg0, deg1, x, W1l, b1l.reshape(1, H), W1r, y2d)


def _layer2_body(p0_ref, p1_ref, d0_ref, d1_ref, h_ref, wl_ref, bl_ref,
                 wr_ref, psim_ref, out_ref):
    deg = jnp.maximum(d0_ref[...] + d1_ref[...], 1.0)
    mean = (p0_ref[...] + p1_ref[...]) / deg
    h = h_ref[...]
    z = (
        lax.dot_general(mean, wl_ref[...], (((1,), (1,)), ((), ())),
                        preferred_element_type=jnp.float32)
        + bl_ref[...]
        + lax.dot_general(h, wr_ref[...], (((1,), (1,)), ((), ())),
                          preferred_element_type=jnp.float32)
    )
    lc = _elu(z)
    m = jnp.max(lc, axis=1, keepdims=True)
    p_lc = lc - m - jnp.log(jnp.sum(jnp.exp(lc - m), axis=1, keepdims=True))
    out_ref[...] = ETA * p_lc + (1.0 - ETA) * psim_ref[...]


def _layer2(parts0, parts1, deg0, deg1, h, W2l, b2l, W2r, p_sim):
    R = 1000
    grid = N // R
    return pl.pallas_call(
        _layer2_body,
        grid=(grid,),
        in_specs=[
            pl.BlockSpec((R, H), lambda i: (i, 0)),
            pl.BlockSpec((R, H), lambda i: (i, 0)),
            pl.BlockSpec((R, 1), lambda i: (i, 0)),
            pl.BlockSpec((R, 1), lambda i: (i, 0)),
            pl.BlockSpec((R, H), lambda i: (i, 0)),
            pl.BlockSpec((C, H), lambda i: (0, 0)),
            pl.BlockSpec((1, C), lambda i: (0, 0)),
            pl.BlockSpec((C, H), lambda i: (0, 0)),
            pl.BlockSpec((R, C), lambda i: (i, 0)),
        ],
        out_specs=pl.BlockSpec((R, C), lambda i: (i, 0)),
        out_shape=jax.ShapeDtypeStruct((N, C), jnp.float32),
        interpret=_INTERPRET,
    )(parts0, parts1, deg0, deg1, h, W2l, b2l.reshape(1, C), W2r, p_sim)


_RB = 200


_GW = 1280


def _fuse_body(hn_full_ref, hn_blk_ref, mask_ref, oh_ref, out_ref, t7_ref):
    hnb = hn_blk_ref[...]
    sm = []
    for g in range(8):
        w0 = _GW * g
        wd = _GW if g < 7 else N - 7 * _GW
        sg = lax.dot_general(hnb, hn_full_ref[w0:w0 + wd, :],
                             (((1,), (1,)), ((), ())),
                             preferred_element_type=jnp.float32)
        sm.append(sg * mask_ref[:, w0:w0 + wd])

    t7_ref[...] = jnp.full(t7_ref.shape, -jnp.inf, jnp.float32)
    t7_ref[:, :N - 7 * _GW] = sm[7]
    members = sm[:7] + [t7_ref[...]]

    t1 = functools.reduce(jnp.maximum, members)
    t2 = functools.reduce(
        jnp.maximum,
        [jnp.where(mg == t1, -jnp.inf, mg) for mg in members])

    def step(_, carry):
        c1, c2 = carry
        m = jnp.max(c1, axis=1, keepdims=True)
        win = c1 >= m
        return jnp.where(win, c2, c1), jnp.where(win, -jnp.inf, c2)

    t1f, _ = lax.fori_loop(0, K - 1, step, (t1, t2))
    thresh = jnp.max(t1f, axis=1, keepdims=True)

    fuse = jnp.zeros((out_ref.shape[0], C), jnp.float32)
    for g in range(8):
        w0 = _GW * g
        wd = _GW if g < 7 else N - 7 * _GW
        wg = jnp.where(sm[g] >= thresh, jnp.exp(sm[g]), 0.0)
        fuse = fuse + lax.dot_general(wg, oh_ref[w0:w0 + wd, :],
                                      (((1,), (0,)), ((), ())),
                                      preferred_element_type=jnp.float32)
    m = jnp.max(fuse, axis=1, keepdims=True)
    out_ref[...] = fuse - m - jnp.log(jnp.sum(jnp.exp(fuse - m), axis=1,
                                              keepdims=True))


def _fused_sim(hn, mask, onehot):
    grid = N // _RB
    return pl.pallas_call(
        _fuse_body,
        grid=(grid,),
        in_specs=[
            pl.BlockSpec((N, H), lambda i: (0, 0)),
            pl.BlockSpec((_RB, H), lambda i: (i, 0)),
            pl.BlockSpec((_RB, N), lambda i: (i, 0)),
            pl.BlockSpec((N, C), lambda i: (0, 0)),
        ],
        out_specs=pl.BlockSpec((_RB, C), lambda i: (i, 0)),
        out_shape=jax.ShapeDtypeStruct((N, C), jnp.float32),
        scratch_shapes=[pltpu.VMEM((_RB, _GW), jnp.float32)],
        interpret=_INTERPRET,
    )(hn, hn, mask, onehot)


_NB = 1280
_BPW = _NB // 32
_NPAD = 10240
_RPS = _NPAD // 16


def _make_sc_segsum(with_deg):
    mesh = plsc.VectorSubcoreMesh(core_axis_name="c", subcore_axis_name="s")
    out_type = [jax.ShapeDtypeStruct((2, _NPAD, D), jnp.float32)]
    scratch = [
        pltpu.VMEM((_BPW, 128), jnp.int32),
        pltpu.VMEM((_BPW, 128), jnp.int32),
        pltpu.VMEM((128, D), jnp.float32),
        pltpu.VMEM((32, D), jnp.float32),
        pltpu.VMEM_SHARED((_NPAD, D), jnp.float32),
        pltpu.SemaphoreType.DMA,
    ]
    if with_deg:
        out_type.append(jax.ShapeDtypeStruct((2, _NPAD // 128, 128),
                                             jnp.float32))
        scratch += [
            pltpu.VMEM((1, 128), jnp.int32),
            pltpu.VMEM((1, 128), jnp.int32),
            pltpu.VMEM((128, D), jnp.float32),
            pltpu.VMEM_SHARED((_NPAD // 128, 128), jnp.float32),
        ]

    @functools.partial(pl.kernel, out_type=out_type, mesh=mesh,
                       scratch_types=scratch)
    def segsum(*args):
        if with_deg:
            (feat, src2d, dst2d, eye, out, dout,
             sidx, didx, rows, zbuf, acc, sem, dmrow, ddrow, orows,
             dacc) = args
        else:
            feat, src2d, dst2d, out, sidx, didx, rows, zbuf, acc, sem = args
        c = lax.axis_index("c")
        s = lax.axis_index("s")
        w = c * 16 + s

        def zrow(r, carry):
            for j in range(D // 16):
                zbuf[r, pl.ds(j * 16, 16)] = jnp.zeros((16,), jnp.float32)
            return carry

        lax.fori_loop(0, 32, zrow, 0)
        for i in range(20):
            pltpu.sync_copy(zbuf, acc.at[pl.ds(s * _RPS + i * 32, 32)])
        if with_deg:
            @pl.when(s == 0)
            def _():
                for i in range(_NPAD // 128 // 16):
                    pltpu.sync_copy(zbuf.at[pl.ds(0, 16)],
                                    dacc.at[pl.ds(i * 16, 16)])
        plsc.subcore_barrier()

        pltpu.sync_copy(src2d.at[pl.ds(w * _BPW, _BPW)], sidx)
        pltpu.sync_copy(dst2d.at[pl.ds(w * _BPW, _BPW)], didx)

        def batch(j, carry):
            if with_deg:
                cp_f = pltpu.async_copy(feat.at[sidx.at[j]], rows, sem)
                for k in range(8):
                    dv = didx[j, pl.ds(k * 16, 16)]
                    dmrow[0, pl.ds(k * 16, 16)] = lax.rem(dv, 128)
                    ddrow[0, pl.ds(k * 16, 16)] = lax.div(dv, 128)
                cp_e = pltpu.async_copy(eye.at[dmrow.at[0]], orows, sem)
                cp_f.wait()
                cp_e.wait()
                pltpu.sync_copy(rows, acc.at[didx.at[j]], add=True)
                pltpu.sync_copy(orows, dacc.at[ddrow.at[0]], add=True)
            else:
                pltpu.async_copy(feat.at[sidx.at[j]], rows, sem).wait()
                pltpu.sync_copy(rows, acc.at[didx.at[j]], add=True)
            return carry

        lax.fori_loop(0, _BPW, batch, 0)
        plsc.subcore_barrier()

        for i in range(20):
            r0 = s * _RPS + i * 32
            pltpu.sync_copy(acc.at[pl.ds(r0, 32)], zbuf)
            pltpu.sync_copy(zbuf, out.at[c].at[pl.ds(r0, 32)])
        if with_deg:
            @pl.when(s == 1)
            def _():
                pltpu.sync_copy(dacc, rows.at[pl.ds(0, _NPAD // 128)])
                pltpu.sync_copy(rows.at[pl.ds(0, _NPAD // 128)], dout.at[c])

    return segsum


_sc_segsum_deg = _make_sc_segsum(True)
_sc_segsum = _make_sc_segsum(False)


def kernel(x, edge_index, y, mask, W1l, b1l, W1r, W2l, b2l, W2r):
    npad = _NB * 128 - E
    src2d = jnp.concatenate(
        [edge_index[0], jnp.zeros((npad,), jnp.int32)]).reshape(_NB, 128)
    dst_flat = jnp.concatenate(
        [edge_index[1], jnp.full((npad,), N + 16, jnp.int32)])
    dst2d = dst_flat.reshape(_NB, 128)
    eye = jnp.eye(128, dtype=jnp.float32)

    parts1, degp = _sc_segsum_deg(x, src2d, dst2d, eye)
    deg0 = degp[0].reshape(_NPAD, 1)
    deg1 = degp[1].reshape(_NPAD, 1)
    h, hn, onehot = _layer1(parts1[0], parts1[1], deg0, deg1, x, W1l, b1l,
                            W1r, y.reshape(N, 1))

    (parts2,) = _sc_segsum(h, src2d, dst2d)
    p_sim = _fused_sim(hn, mask, onehot)
    final = _layer2(parts2[0], parts2[1], deg0, deg1, h, W2l, b2l, W2r, p_sim)
    return (final, h)


# speedup vs baseline: 11.9081x; 1.0210x over previous
import functools

import jax
import jax.numpy as jnp
from jax import lax
from jax.experimental import pallas as pl
from jax.experimental.pallas import tpu as pltpu
from jax.experimental.pallas import tpu_sc as plsc

N = 10000
E = 160000
D = 128
H = 128
C = 64
K = 16
ETA = 0.5

_INTERPRET = False


def _elu(v):
    return jnp.where(v > 0, v, jnp.exp(v) - 1.0)


def _layer1_body(p0_ref, p1_ref, d0_ref, d1_ref, x_ref, wl_ref, bl_ref,
                 wr_ref, y_ref, h_ref, hn_ref, oh_ref):
    deg = jnp.maximum(d0_ref[...] + d1_ref[...], 1.0)
    mean = (p0_ref[...] + p1_ref[...]) / deg
    z = (
        lax.dot_general(mean, wl_ref[...], (((1,), (1,)), ((), ())),
                        preferred_element_type=jnp.float32)
        + bl_ref[...]
        + lax.dot_general(x_ref[...], wr_ref[...], (((1,), (1,)), ((), ())),
                          preferred_element_type=jnp.float32)
    )
    h = _elu(z)
    h_ref[...] = h
    nrm = jnp.sqrt(jnp.sum(h * h, axis=1, keepdims=True))
    hn_ref[...] = h / jnp.maximum(nrm, 1e-8)
    cls = lax.broadcasted_iota(jnp.int32, oh_ref.shape, 1)
    oh_ref[...] = jnp.where(y_ref[...] == cls, 1.0, 0.0)


def _layer1(parts0, parts1, deg0, deg1, x, W1l, b1l, W1r, y2d):
    R = 1000
    grid = N // R
    return pl.pallas_call(
        _layer1_body,
        grid=(grid,),
        in_specs=[
            pl.BlockSpec((R, D), lambda i: (i, 0)),
            pl.BlockSpec((R, D), lambda i: (i, 0)),
            pl.BlockSpec((R, 1), lambda i: (i, 0)),
            pl.BlockSpec((R, 1), lambda i: (i, 0)),
            pl.BlockSpec((R, D), lambda i: (i, 0)),
            pl.BlockSpec((H, D), lambda i: (0, 0)),
            pl.BlockSpec((1, H), lambda i: (0, 0)),
            pl.BlockSpec((H, D), lambda i: (0, 0)),
            pl.BlockSpec((R, 1), lambda i: (i, 0)),
        ],
        out_specs=[
            pl.BlockSpec((R, H), lambda i: (i, 0)),
            pl.BlockSpec((R, H), lambda i: (i, 0)),
            pl.BlockSpec((R, C), lambda i: (i, 0)),
        ],
        out_shape=[
            jax.ShapeDtypeStruct((N, H), jnp.float32),
            jax.ShapeDtypeStruct((N, H), jnp.float32),
            jax.ShapeDtypeStruct((N, C), jnp.float32),
        ],
        interpret=_INTERPRET,
    )(parts0, parts1, deg0, deg1, x, W1l, b1l.reshape(1, H), W1r, y2d)


def _layer2_body(p0_ref, p1_ref, d0_ref, d1_ref, h_ref, wl_ref, bl_ref,
                 wr_ref, psim_ref, out_ref):
    deg = jnp.maximum(d0_ref[...] + d1_ref[...], 1.0)
    mean = (p0_ref[...] + p1_ref[...]) / deg
    h = h_ref[...]
    z = (
        lax.dot_general(mean, wl_ref[...], (((1,), (1,)), ((), ())),
                        preferred_element_type=jnp.float32)
        + bl_ref[...]
        + lax.dot_general(h, wr_ref[...], (((1,), (1,)), ((), ())),
                          preferred_element_type=jnp.float32)
    )
    lc = _elu(z)
    m = jnp.max(lc, axis=1, keepdims=True)
    p_lc = lc - m - jnp.log(jnp.sum(jnp.exp(lc - m), axis=1, keepdims=True))
    out_ref[...] = ETA * p_lc + (1.0 - ETA) * psim_ref[...]


def _layer2(parts0, parts1, deg0, deg1, h, W2l, b2l, W2r, p_sim):
    R = 1000
    grid = N // R
    return pl.pallas_call(
        _layer2_body,
        grid=(grid,),
        in_specs=[
            pl.BlockSpec((R, H), lambda i: (i, 0)),
            pl.BlockSpec((R, H), lambda i: (i, 0)),
            pl.BlockSpec((R, 1), lambda i: (i, 0)),
            pl.BlockSpec((R, 1), lambda i: (i, 0)),
            pl.BlockSpec((R, H), lambda i: (i, 0)),
            pl.BlockSpec((C, H), lambda i: (0, 0)),
            pl.BlockSpec((1, C), lambda i: (0, 0)),
            pl.BlockSpec((C, H), lambda i: (0, 0)),
            pl.BlockSpec((R, C), lambda i: (i, 0)),
        ],
        out_specs=pl.BlockSpec((R, C), lambda i: (i, 0)),
        out_shape=jax.ShapeDtypeStruct((N, C), jnp.float32),
        interpret=_INTERPRET,
    )(parts0, parts1, deg0, deg1, h, W2l, b2l.reshape(1, C), W2r, p_sim)


_RB = 200


_GW = 1280


def _fuse_body(hn_full_ref, hn_blk_ref, mask_ref, oh_ref, out_ref, t7_ref):
    hnb = hn_blk_ref[...]
    sm = []
    for g in range(8):
        w0 = _GW * g
        wd = _GW if g < 7 else N - 7 * _GW
        sg = lax.dot_general(hnb, hn_full_ref[w0:w0 + wd, :],
                             (((1,), (1,)), ((), ())),
                             preferred_element_type=jnp.float32)
        sm.append(sg * mask_ref[:, w0:w0 + wd])

    t7_ref[...] = jnp.full(t7_ref.shape, -jnp.inf, jnp.float32)
    t7_ref[:, :N - 7 * _GW] = sm[7]
    members = sm[:7] + [t7_ref[...]]

    t1 = functools.reduce(jnp.maximum, members)
    t2 = functools.reduce(
        jnp.maximum,
        [jnp.where(mg == t1, -jnp.inf, mg) for mg in members])

    def step(_, carry):
        c1, c2 = carry
        m = jnp.max(c1, axis=1, keepdims=True)
        win = c1 >= m
        return jnp.where(win, c2, c1), jnp.where(win, -jnp.inf, c2)

    t1f, _ = lax.fori_loop(0, K - 1, step, (t1, t2))
    thresh = jnp.max(t1f, axis=1, keepdims=True)

    fuse = jnp.zeros((out_ref.shape[0], C), jnp.float32)
    for g in range(8):
        w0 = _GW * g
        wd = _GW if g < 7 else N - 7 * _GW
        wg = jnp.where(sm[g] >= thresh, jnp.exp(sm[g]), 0.0)
        fuse = fuse + lax.dot_general(wg, oh_ref[w0:w0 + wd, :],
                                      (((1,), (0,)), ((), ())),
                                      preferred_element_type=jnp.float32)
    m = jnp.max(fuse, axis=1, keepdims=True)
    out_ref[...] = fuse - m - jnp.log(jnp.sum(jnp.exp(fuse - m), axis=1,
                                              keepdims=True))


def _fused_sim(hn, mask, onehot):
    grid = N // _RB
    return pl.pallas_call(
        _fuse_body,
        grid=(grid,),
        in_specs=[
            pl.BlockSpec((N, H), lambda i: (0, 0)),
            pl.BlockSpec((_RB, H), lambda i: (i, 0)),
            pl.BlockSpec((_RB, N), lambda i: (i, 0)),
            pl.BlockSpec((N, C), lambda i: (0, 0)),
        ],
        out_specs=pl.BlockSpec((_RB, C), lambda i: (i, 0)),
        out_shape=jax.ShapeDtypeStruct((N, C), jnp.float32),
        scratch_shapes=[pltpu.VMEM((_RB, _GW), jnp.float32)],
        interpret=_INTERPRET,
    )(hn, hn, mask, onehot)


_NB = 1280
_BPW = _NB // 32
_NPAD = 10240
_RPS = _NPAD // 16


def _make_sc_segsum(with_deg):
    mesh = plsc.VectorSubcoreMesh(core_axis_name="c", subcore_axis_name="s")
    out_type = [jax.ShapeDtypeStruct((2, _NPAD, D), jnp.float32)]
    nbuf = 1 if with_deg else 2
    scratch = [
        pltpu.VMEM((_BPW, 128), jnp.int32),
        pltpu.VMEM((_BPW, 128), jnp.int32),
        pltpu.VMEM((nbuf, 128, D), jnp.float32),
        pltpu.VMEM((32, D), jnp.float32),
        pltpu.VMEM_SHARED((_NPAD, D), jnp.float32),
        pltpu.SemaphoreType.DMA,
        pltpu.SemaphoreType.DMA,
    ]
    if with_deg:
        out_type.append(jax.ShapeDtypeStruct((2, _NPAD // 128, 128),
                                             jnp.float32))
        scratch += [
            pltpu.VMEM((1, 128), jnp.int32),
            pltpu.VMEM((1, 128), jnp.int32),
            pltpu.VMEM((128, D), jnp.float32),
            pltpu.VMEM_SHARED((_NPAD // 128, 128), jnp.float32),
            pltpu.SemaphoreType.DMA,
        ]

    @functools.partial(pl.kernel, out_type=out_type, mesh=mesh,
                       scratch_types=scratch)
    def segsum(*args):
        if with_deg:
            (feat, src2d, dst2d, eye, out, dout,
             sidx, didx, rows, zbuf, acc, semg, sems, dmrow, ddrow, orows,
             dacc, seme) = args
        else:
            (feat, src2d, dst2d, out,
             sidx, didx, rows, zbuf, acc, semg, sems) = args
        c = lax.axis_index("c")
        s = lax.axis_index("s")
        w = c * 16 + s

        def zrow(r, carry):
            for j in range(D // 16):
                zbuf[r, pl.ds(j * 16, 16)] = jnp.zeros((16,), jnp.float32)
            return carry

        lax.fori_loop(0, 32, zrow, 0)
        for i in range(20):
            pltpu.sync_copy(zbuf, acc.at[pl.ds(s * _RPS + i * 32, 32)])
        if with_deg:
            @pl.when(s == 0)
            def _():
                for i in range(_NPAD // 128 // 16):
                    pltpu.sync_copy(zbuf.at[pl.ds(0, 16)],
                                    dacc.at[pl.ds(i * 16, 16)])
        plsc.subcore_barrier()

        pltpu.sync_copy(src2d.at[pl.ds(w * _BPW, _BPW)], sidx)
        pltpu.sync_copy(dst2d.at[pl.ds(w * _BPW, _BPW)], didx)

        if with_deg:
            buf = rows.at[0]
            pltpu.async_copy(feat.at[sidx.at[0]], buf, semg)

            def batch(j, carry):
                pltpu.make_async_copy(feat.at[sidx.at[j]], buf, semg).wait()
                pltpu.async_copy(buf, acc.at[didx.at[j]], sems, add=True)
                for k in range(8):
                    dv = didx[j, pl.ds(k * 16, 16)]
                    dmrow[0, pl.ds(k * 16, 16)] = lax.rem(dv, 128)
                    ddrow[0, pl.ds(k * 16, 16)] = lax.div(dv, 128)
                pltpu.async_copy(eye.at[dmrow.at[0]], orows, seme)
                pltpu.make_async_copy(buf, acc.at[didx.at[j]], sems).wait()

                @pl.when(j + 1 < _BPW)
                def _g():
                    pltpu.async_copy(feat.at[sidx.at[j + 1]], buf, semg)

                pltpu.make_async_copy(eye.at[dmrow.at[0]], orows, seme).wait()
                pltpu.sync_copy(orows, dacc.at[ddrow.at[0]], add=True)
                return carry

            lax.fori_loop(0, _BPW, batch, 0)
        else:
            pltpu.async_copy(feat.at[sidx.at[0]], rows.at[0], semg)

            def halfstep(j, b):
                pltpu.make_async_copy(feat.at[sidx.at[j]], rows.at[b],
                                      semg).wait()

                @pl.when(j >= 1)
                def _sw():
                    pltpu.make_async_copy(rows.at[1 - b],
                                          acc.at[didx.at[j - 1]], sems).wait()

                @pl.when(j + 1 < _BPW)
                def _gs():
                    pltpu.async_copy(feat.at[sidx.at[j + 1]], rows.at[1 - b],
                                     semg)

                pltpu.async_copy(rows.at[b], acc.at[didx.at[j]], sems,
                                 add=True)

            def pair(p, carry):
                halfstep(2 * p, 0)
                halfstep(2 * p + 1, 1)
                return carry

            lax.fori_loop(0, _BPW // 2, pair, 0)
            pltpu.make_async_copy(rows.at[1], acc.at[didx.at[_BPW - 1]],
                                  sems).wait()
        plsc.subcore_barrier()

        for i in range(20):
            r0 = s * _RPS + i * 32
            pltpu.sync_copy(acc.at[pl.ds(r0, 32)], zbuf)
            pltpu.sync_copy(zbuf, out.at[c].at[pl.ds(r0, 32)])
        if with_deg:
            @pl.when(s == 1)
            def _():
                pltpu.sync_copy(dacc, orows.at[pl.ds(0, _NPAD // 128)])
                pltpu.sync_copy(orows.at[pl.ds(0, _NPAD // 128)], dout.at[c])

    return segsum


_sc_segsum_deg = _make_sc_segsum(True)
_sc_segsum = _make_sc_segsum(False)


def kernel(x, edge_index, y, mask, W1l, b1l, W1r, W2l, b2l, W2r):
    npad = _NB * 128 - E
    src2d = jnp.concatenate(
        [edge_index[0], jnp.zeros((npad,), jnp.int32)]).reshape(_NB, 128)
    dst_flat = jnp.concatenate(
        [edge_index[1], jnp.full((npad,), N + 16, jnp.int32)])
    dst2d = dst_flat.reshape(_NB, 128)
    eye = jnp.eye(128, dtype=jnp.float32)

    parts1, degp = _sc_segsum_deg(x, src2d, dst2d, eye)
    deg0 = degp[0].reshape(_NPAD, 1)
    deg1 = degp[1].reshape(_NPAD, 1)
    h, hn, onehot = _layer1(parts1[0], parts1[1], deg0, deg1, x, W1l, b1l,
                            W1r, y.reshape(N, 1))

    (parts2,) = _sc_segsum(h, src2d, dst2d)
    p_sim = _fused_sim(hn, mask, onehot)
    final = _layer2(parts2[0], parts2[1], deg0, deg1, h, W2l, b2l, W2r, p_sim)
    return (final, h)


# speedup vs baseline: 12.1529x; 1.0206x over previous
import functools

import jax
import jax.numpy as jnp
from jax import lax
from jax.experimental import pallas as pl
from jax.experimental.pallas import tpu as pltpu
from jax.experimental.pallas import tpu_sc as plsc

N = 10000
E = 160000
D = 128
H = 128
C = 64
K = 16
ETA = 0.5

_INTERPRET = False


def _elu(v):
    return jnp.where(v > 0, v, jnp.exp(v) - 1.0)


def _layer1_body(p0_ref, p1_ref, d0_ref, d1_ref, x_ref, wl_ref, bl_ref,
                 wr_ref, y_ref, h_ref, hn_ref, oh_ref):
    deg = jnp.maximum(d0_ref[...] + d1_ref[...], 1.0)
    mean = (p0_ref[...] + p1_ref[...]) / deg
    z = (
        lax.dot_general(mean, wl_ref[...], (((1,), (1,)), ((), ())),
                        preferred_element_type=jnp.float32)
        + bl_ref[...]
        + lax.dot_general(x_ref[...], wr_ref[...], (((1,), (1,)), ((), ())),
                          preferred_element_type=jnp.float32)
    )
    h = _elu(z)
    h_ref[...] = h
    nrm = jnp.sqrt(jnp.sum(h * h, axis=1, keepdims=True))
    hn_ref[...] = h / jnp.maximum(nrm, 1e-8)
    cls = lax.broadcasted_iota(jnp.int32, oh_ref.shape, 1)
    oh_ref[...] = jnp.where(y_ref[...] == cls, 1.0, 0.0)


def _layer1(parts0, parts1, deg0, deg1, x, W1l, b1l, W1r, y2d):
    R = 1000
    grid = N // R
    return pl.pallas_call(
        _layer1_body,
        grid=(grid,),
        in_specs=[
            pl.BlockSpec((R, D), lambda i: (i, 0)),
            pl.BlockSpec((R, D), lambda i: (i, 0)),
            pl.BlockSpec((R, 1), lambda i: (i, 0)),
            pl.BlockSpec((R, 1), lambda i: (i, 0)),
            pl.BlockSpec((R, D), lambda i: (i, 0)),
            pl.BlockSpec((H, D), lambda i: (0, 0)),
            pl.BlockSpec((1, H), lambda i: (0, 0)),
            pl.BlockSpec((H, D), lambda i: (0, 0)),
            pl.BlockSpec((R, 1), lambda i: (i, 0)),
        ],
        out_specs=[
            pl.BlockSpec((R, H), lambda i: (i, 0)),
            pl.BlockSpec((R, H), lambda i: (i, 0)),
            pl.BlockSpec((R, C), lambda i: (i, 0)),
        ],
        out_shape=[
            jax.ShapeDtypeStruct((N, H), jnp.float32),
            jax.ShapeDtypeStruct((N, H), jnp.float32),
            jax.ShapeDtypeStruct((N, C), jnp.float32),
        ],
        interpret=_INTERPRET,
    )(parts0, parts1, deg0, deg1, x, W1l, b1l.reshape(1, H), W1r, y2d)


def _layer2_body(p0_ref, p1_ref, d0_ref, d1_ref, h_ref, wl_ref, bl_ref,
                 wr_ref, psim_ref, out_ref):
    deg = jnp.maximum(d0_ref[...] + d1_ref[...], 1.0)
    mean = (p0_ref[...] + p1_ref[...]) / deg
    h = h_ref[...]
    z = (
        lax.dot_general(mean, wl_ref[...], (((1,), (1,)), ((), ())),
                        preferred_element_type=jnp.float32)
        + bl_ref[...]
        + lax.dot_general(h, wr_ref[...], (((1,), (1,)), ((), ())),
                          preferred_element_type=jnp.float32)
    )
    lc = _elu(z)
    m = jnp.max(lc, axis=1, keepdims=True)
    p_lc = lc - m - jnp.log(jnp.sum(jnp.exp(lc - m), axis=1, keepdims=True))
    out_ref[...] = ETA * p_lc + (1.0 - ETA) * psim_ref[...]


def _layer2(parts0, parts1, deg0, deg1, h, W2l, b2l, W2r, p_sim):
    R = 1000
    grid = N // R
    return pl.pallas_call(
        _layer2_body,
        grid=(grid,),
        in_specs=[
            pl.BlockSpec((R, H), lambda i: (i, 0)),
            pl.BlockSpec((R, H), lambda i: (i, 0)),
            pl.BlockSpec((R, 1), lambda i: (i, 0)),
            pl.BlockSpec((R, 1), lambda i: (i, 0)),
            pl.BlockSpec((R, H), lambda i: (i, 0)),
            pl.BlockSpec((C, H), lambda i: (0, 0)),
            pl.BlockSpec((1, C), lambda i: (0, 0)),
            pl.BlockSpec((C, H), lambda i: (0, 0)),
            pl.BlockSpec((R, C), lambda i: (i, 0)),
        ],
        out_specs=pl.BlockSpec((R, C), lambda i: (i, 0)),
        out_shape=jax.ShapeDtypeStruct((N, C), jnp.float32),
        interpret=_INTERPRET,
    )(parts0, parts1, deg0, deg1, h, W2l, b2l.reshape(1, C), W2r, p_sim)


_RB = 200


_GW = 1280


def _fuse_body(hn_full_ref, hn_blk_ref, mask_ref, oh_ref, out_ref, t7_ref):
    hnb = hn_blk_ref[...]
    sm = []
    for g in range(8):
        w0 = _GW * g
        wd = _GW if g < 7 else N - 7 * _GW
        sg = lax.dot_general(hnb, hn_full_ref[w0:w0 + wd, :],
                             (((1,), (1,)), ((), ())),
                             preferred_element_type=jnp.float32)
        sm.append(sg * mask_ref[:, w0:w0 + wd])

    t7_ref[...] = jnp.full(t7_ref.shape, -jnp.inf, jnp.float32)
    t7_ref[:, :N - 7 * _GW] = sm[7]
    members = sm[:7] + [t7_ref[...]]

    t1 = functools.reduce(jnp.maximum, members)
    t2 = functools.reduce(
        jnp.maximum,
        [jnp.where(mg == t1, -jnp.inf, mg) for mg in members])

    def step(_, carry):
        c1, c2 = carry
        m = jnp.max(c1, axis=1, keepdims=True)
        win = c1 >= m
        return jnp.where(win, c2, c1), jnp.where(win, -jnp.inf, c2)

    t1f, _ = lax.fori_loop(0, K - 1, step, (t1, t2))
    thresh = jnp.max(t1f, axis=1, keepdims=True)

    fuse = jnp.zeros((out_ref.shape[0], C), jnp.float32)
    for g in range(8):
        w0 = _GW * g
        wd = _GW if g < 7 else N - 7 * _GW
        wg = jnp.where(sm[g] >= thresh, jnp.exp(sm[g]), 0.0)
        fuse = fuse + lax.dot_general(wg, oh_ref[w0:w0 + wd, :],
                                      (((1,), (0,)), ((), ())),
                                      preferred_element_type=jnp.float32)
    m = jnp.max(fuse, axis=1, keepdims=True)
    out_ref[...] = fuse - m - jnp.log(jnp.sum(jnp.exp(fuse - m), axis=1,
                                              keepdims=True))


def _fused_sim(hn, mask, onehot):
    grid = N // _RB
    return pl.pallas_call(
        _fuse_body,
        grid=(grid,),
        in_specs=[
            pl.BlockSpec((N, H), lambda i: (0, 0)),
            pl.BlockSpec((_RB, H), lambda i: (i, 0)),
            pl.BlockSpec((_RB, N), lambda i: (i, 0)),
            pl.BlockSpec((N, C), lambda i: (0, 0)),
        ],
        out_specs=pl.BlockSpec((_RB, C), lambda i: (i, 0)),
        out_shape=jax.ShapeDtypeStruct((N, C), jnp.float32),
        scratch_shapes=[pltpu.VMEM((_RB, _GW), jnp.float32)],
        interpret=_INTERPRET,
    )(hn, hn, mask, onehot)


_NB = 1280
_BPW = _NB // 32
_NPAD = 10240
_RPS = _NPAD // 16


def _make_sc_segsum(with_deg):
    mesh = plsc.VectorSubcoreMesh(core_axis_name="c", subcore_axis_name="s")
    out_type = [jax.ShapeDtypeStruct((2, _NPAD, D), jnp.float32)]
    nbuf = 1 if with_deg else 2
    scratch = [
        pltpu.VMEM((_BPW, 128), jnp.int32),
        pltpu.VMEM((_BPW, 128), jnp.int32),
        pltpu.VMEM((nbuf, 128, D), jnp.float32),
        pltpu.VMEM((32, D), jnp.float32),
        pltpu.VMEM_SHARED((_NPAD, D), jnp.float32),
        pltpu.SemaphoreType.DMA,
        pltpu.SemaphoreType.DMA,
    ]
    if with_deg:
        out_type.append(jax.ShapeDtypeStruct((2, _NPAD // 128, 128),
                                             jnp.float32))
        scratch += [
            pltpu.VMEM((1, 128), jnp.int32),
            pltpu.VMEM((1, 128), jnp.int32),
            pltpu.VMEM((128, D), jnp.float32),
            pltpu.VMEM_SHARED((_NPAD // 128, 128), jnp.float32),
            pltpu.VMEM_SHARED((128, 128), jnp.float32),
            pltpu.SemaphoreType.DMA,
        ]

    @functools.partial(pl.kernel, out_type=out_type, mesh=mesh,
                       scratch_types=scratch)
    def segsum(*args):
        if with_deg:
            (feat, src2d, dst2d, eye, out, dout,
             sidx, didx, rows, zbuf, acc, semg, sems, dmrow, ddrow, orows,
             dacc, eyespm, seme) = args
        else:
            (feat, src2d, dst2d, out,
             sidx, didx, rows, zbuf, acc, semg, sems) = args
        c = lax.axis_index("c")
        s = lax.axis_index("s")
        w = c * 16 + s

        def zrow(r, carry):
            for j in range(D // 16):
                zbuf[r, pl.ds(j * 16, 16)] = jnp.zeros((16,), jnp.float32)
            return carry

        lax.fori_loop(0, 32, zrow, 0)
        for i in range(20):
            pltpu.sync_copy(zbuf, acc.at[pl.ds(s * _RPS + i * 32, 32)])
        if with_deg:
            @pl.when(s == 0)
            def _():
                for i in range(_NPAD // 128 // 16):
                    pltpu.sync_copy(zbuf.at[pl.ds(0, 16)],
                                    dacc.at[pl.ds(i * 16, 16)])

            @pl.when(s == 1)
            def _():
                pltpu.sync_copy(eye, orows)
                pltpu.sync_copy(orows, eyespm)
        plsc.subcore_barrier()

        pltpu.sync_copy(src2d.at[pl.ds(w * _BPW, _BPW)], sidx)
        pltpu.sync_copy(dst2d.at[pl.ds(w * _BPW, _BPW)], didx)

        if with_deg:
            buf = rows.at[0]
            pltpu.async_copy(feat.at[sidx.at[0]], buf, semg)

            def batch(j, carry):
                pltpu.make_async_copy(feat.at[sidx.at[j]], buf, semg).wait()
                pltpu.async_copy(buf, acc.at[didx.at[j]], sems, add=True)
                for k in range(8):
                    dv = didx[j, pl.ds(k * 16, 16)]
                    dmrow[0, pl.ds(k * 16, 16)] = lax.rem(dv, 128)
                    ddrow[0, pl.ds(k * 16, 16)] = lax.div(dv, 128)
                pltpu.async_copy(eyespm.at[dmrow.at[0]], orows, seme)
                pltpu.make_async_copy(buf, acc.at[didx.at[j]], sems).wait()

                @pl.when(j + 1 < _BPW)
                def _g():
                    pltpu.async_copy(feat.at[sidx.at[j + 1]], buf, semg)

                pltpu.make_async_copy(eyespm.at[dmrow.at[0]], orows,
                                      seme).wait()
                pltpu.sync_copy(orows, dacc.at[ddrow.at[0]], add=True)
                return carry

            lax.fori_loop(0, _BPW, batch, 0)
        else:
            pltpu.async_copy(feat.at[sidx.at[0]], rows.at[0], semg)

            def halfstep(j, b):
                pltpu.make_async_copy(feat.at[sidx.at[j]], rows.at[b],
                                      semg).wait()

                @pl.when(j >= 1)
                def _sw():
                    pltpu.make_async_copy(rows.at[1 - b],
                                          acc.at[didx.at[j - 1]], sems).wait()

                @pl.when(j + 1 < _BPW)
                def _gs():
                    pltpu.async_copy(feat.at[sidx.at[j + 1]], rows.at[1 - b],
                                     semg)

                pltpu.async_copy(rows.at[b], acc.at[didx.at[j]], sems,
                                 add=True)

            def pair(p, carry):
                halfstep(2 * p, 0)
                halfstep(2 * p + 1, 1)
                return carry

            lax.fori_loop(0, _BPW // 2, pair, 0)
            pltpu.make_async_copy(rows.at[1], acc.at[didx.at[_BPW - 1]],
                                  sems).wait()
        plsc.subcore_barrier()

        for i in range(20):
            r0 = s * _RPS + i * 32
            pltpu.sync_copy(acc.at[pl.ds(r0, 32)], zbuf)
            pltpu.sync_copy(zbuf, out.at[c].at[pl.ds(r0, 32)])
        if with_deg:
            @pl.when(s == 1)
            def _():
                pltpu.sync_copy(dacc, orows.at[pl.ds(0, _NPAD // 128)])
                pltpu.sync_copy(orows.at[pl.ds(0, _NPAD // 128)], dout.at[c])

    return segsum


_sc_segsum_deg = _make_sc_segsum(True)
_sc_segsum = _make_sc_segsum(False)


def kernel(x, edge_index, y, mask, W1l, b1l, W1r, W2l, b2l, W2r):
    npad = _NB * 128 - E
    src2d = jnp.concatenate(
        [edge_index[0], jnp.zeros((npad,), jnp.int32)]).reshape(_NB, 128)
    dst_flat = jnp.concatenate(
        [edge_index[1], jnp.full((npad,), N + 16, jnp.int32)])
    dst2d = dst_flat.reshape(_NB, 128)
    eye = jnp.eye(128, dtype=jnp.float32)

    parts1, degp = _sc_segsum_deg(x, src2d, dst2d, eye)
    deg0 = degp[0].reshape(_NPAD, 1)
    deg1 = degp[1].reshape(_NPAD, 1)
    h, hn, onehot = _layer1(parts1[0], parts1[1], deg0, deg1, x, W1l, b1l,
                            W1r, y.reshape(N, 1))

    (parts2,) = _sc_segsum(h, src2d, dst2d)
    p_sim = _fused_sim(hn, mask, onehot)
    final = _layer2(parts2[0], parts2[1], deg0, deg1, h, W2l, b2l, W2r, p_sim)
    return (final, h)


# speedup vs baseline: 12.1779x; 1.0021x over previous
import functools

import jax
import jax.numpy as jnp
from jax import lax
from jax.experimental import pallas as pl
from jax.experimental.pallas import tpu as pltpu
from jax.experimental.pallas import tpu_sc as plsc

N = 10000
E = 160000
D = 128
H = 128
C = 64
K = 16
ETA = 0.5

_INTERPRET = False


def _elu(v):
    return jnp.where(v > 0, v, jnp.exp(v) - 1.0)


def _layer1_body(p0_ref, p1_ref, d0_ref, d1_ref, x_ref, wl_ref, bl_ref,
                 wr_ref, y_ref, h_ref, hn_ref, oh_ref):
    deg = jnp.maximum(d0_ref[...] + d1_ref[...], 1.0)
    mean = (p0_ref[...] + p1_ref[...]) / deg
    z = (
        lax.dot_general(mean, wl_ref[...], (((1,), (1,)), ((), ())),
                        preferred_element_type=jnp.float32)
        + bl_ref[...]
        + lax.dot_general(x_ref[...], wr_ref[...], (((1,), (1,)), ((), ())),
                          preferred_element_type=jnp.float32)
    )
    h = _elu(z)
    h_ref[...] = h
    nrm = jnp.sqrt(jnp.sum(h * h, axis=1, keepdims=True))
    hn_ref[...] = h / jnp.maximum(nrm, 1e-8)
    cls = lax.broadcasted_iota(jnp.int32, oh_ref.shape, 1)
    oh_ref[...] = jnp.where(y_ref[...] == cls, 1.0, 0.0)


def _layer1(parts0, parts1, deg0, deg1, x, W1l, b1l, W1r, y2d):
    R = 1000
    grid = N // R
    return pl.pallas_call(
        _layer1_body,
        grid=(grid,),
        in_specs=[
            pl.BlockSpec((R, D), lambda i: (i, 0)),
            pl.BlockSpec((R, D), lambda i: (i, 0)),
            pl.BlockSpec((R, 1), lambda i: (i, 0)),
            pl.BlockSpec((R, 1), lambda i: (i, 0)),
            pl.BlockSpec((R, D), lambda i: (i, 0)),
            pl.BlockSpec((H, D), lambda i: (0, 0)),
            pl.BlockSpec((1, H), lambda i: (0, 0)),
            pl.BlockSpec((H, D), lambda i: (0, 0)),
            pl.BlockSpec((R, 1), lambda i: (i, 0)),
        ],
        out_specs=[
            pl.BlockSpec((R, H), lambda i: (i, 0)),
            pl.BlockSpec((R, H), lambda i: (i, 0)),
            pl.BlockSpec((R, C), lambda i: (i, 0)),
        ],
        out_shape=[
            jax.ShapeDtypeStruct((N, H), jnp.float32),
            jax.ShapeDtypeStruct((N, H), jnp.float32),
            jax.ShapeDtypeStruct((N, C), jnp.float32),
        ],
        interpret=_INTERPRET,
    )(parts0, parts1, deg0, deg1, x, W1l, b1l.reshape(1, H), W1r, y2d)


def _layer2_body(p0_ref, p1_ref, d0_ref, d1_ref, h_ref, wl_ref, bl_ref,
                 wr_ref, psim_ref, out_ref):
    deg = jnp.maximum(d0_ref[...] + d1_ref[...], 1.0)
    mean = (p0_ref[...] + p1_ref[...]) / deg
    h = h_ref[...]
    z = (
        lax.dot_general(mean, wl_ref[...], (((1,), (1,)), ((), ())),
                        preferred_element_type=jnp.float32)
        + bl_ref[...]
        + lax.dot_general(h, wr_ref[...], (((1,), (1,)), ((), ())),
                          preferred_element_type=jnp.float32)
    )
    lc = _elu(z)
    m = jnp.max(lc, axis=1, keepdims=True)
    p_lc = lc - m - jnp.log(jnp.sum(jnp.exp(lc - m), axis=1, keepdims=True))
    out_ref[...] = ETA * p_lc + (1.0 - ETA) * psim_ref[...]


def _layer2(parts0, parts1, deg0, deg1, h, W2l, b2l, W2r, p_sim):
    R = 1000
    grid = N // R
    return pl.pallas_call(
        _layer2_body,
        grid=(grid,),
        in_specs=[
            pl.BlockSpec((R, H), lambda i: (i, 0)),
            pl.BlockSpec((R, H), lambda i: (i, 0)),
            pl.BlockSpec((R, 1), lambda i: (i, 0)),
            pl.BlockSpec((R, 1), lambda i: (i, 0)),
            pl.BlockSpec((R, H), lambda i: (i, 0)),
            pl.BlockSpec((C, H), lambda i: (0, 0)),
            pl.BlockSpec((1, C), lambda i: (0, 0)),
            pl.BlockSpec((C, H), lambda i: (0, 0)),
            pl.BlockSpec((R, C), lambda i: (i, 0)),
        ],
        out_specs=pl.BlockSpec((R, C), lambda i: (i, 0)),
        out_shape=jax.ShapeDtypeStruct((N, C), jnp.float32),
        interpret=_INTERPRET,
    )(parts0, parts1, deg0, deg1, h, W2l, b2l.reshape(1, C), W2r, p_sim)


_RB = 200


_GW = 1280


def _fuse_body(hn_full_ref, hn_blk_ref, mask_ref, oh_ref, out_ref, t7_ref):
    hnb = hn_blk_ref[...]
    sm = []
    for g in range(8):
        w0 = _GW * g
        wd = _GW if g < 7 else N - 7 * _GW
        sg = lax.dot_general(hnb, hn_full_ref[w0:w0 + wd, :],
                             (((1,), (1,)), ((), ())),
                             preferred_element_type=jnp.float32)
        sm.append(sg * mask_ref[:, w0:w0 + wd])

    t7_ref[...] = jnp.full(t7_ref.shape, -jnp.inf, jnp.float32)
    t7_ref[:, :N - 7 * _GW] = sm[7]
    members = sm[:7] + [t7_ref[...]]

    t1 = functools.reduce(jnp.maximum, members)
    t2 = functools.reduce(
        jnp.maximum,
        [jnp.where(mg == t1, -jnp.inf, mg) for mg in members])

    def step(_, carry):
        c1, c2 = carry
        m = jnp.max(c1, axis=1, keepdims=True)
        win = c1 >= m
        return jnp.where(win, c2, c1), jnp.where(win, -jnp.inf, c2)

    t1f, _ = lax.fori_loop(0, K - 1, step, (t1, t2))
    thresh = jnp.max(t1f, axis=1, keepdims=True)

    fuse = jnp.zeros((out_ref.shape[0], C), jnp.float32)
    for g in range(8):
        w0 = _GW * g
        wd = _GW if g < 7 else N - 7 * _GW
        wg = jnp.where(sm[g] >= thresh, jnp.exp(sm[g]), 0.0)
        fuse = fuse + lax.dot_general(wg, oh_ref[w0:w0 + wd, :],
                                      (((1,), (0,)), ((), ())),
                                      preferred_element_type=jnp.float32)
    m = jnp.max(fuse, axis=1, keepdims=True)
    out_ref[...] = fuse - m - jnp.log(jnp.sum(jnp.exp(fuse - m), axis=1,
                                              keepdims=True))


def _fused_sim(hn, mask, onehot):
    grid = N // _RB
    return pl.pallas_call(
        _fuse_body,
        grid=(grid,),
        in_specs=[
            pl.BlockSpec((N, H), lambda i: (0, 0)),
            pl.BlockSpec((_RB, H), lambda i: (i, 0)),
            pl.BlockSpec((_RB, N), lambda i: (i, 0)),
            pl.BlockSpec((N, C), lambda i: (0, 0)),
        ],
        out_specs=pl.BlockSpec((_RB, C), lambda i: (i, 0)),
        out_shape=jax.ShapeDtypeStruct((N, C), jnp.float32),
        scratch_shapes=[pltpu.VMEM((_RB, _GW), jnp.float32)],
        interpret=_INTERPRET,
    )(hn, hn, mask, onehot)


_NB = 1280
_NBR = 1296
_B0 = 48
_B1 = 32
_NPAD = 10240
_RPS = _NPAD // 16


def _make_sc_segsum(with_deg):
    mesh = plsc.VectorSubcoreMesh(core_axis_name="c", subcore_axis_name="s")
    out_type = [jax.ShapeDtypeStruct((2, _NPAD, D), jnp.float32)]
    nbuf = 1 if with_deg else 2
    scratch = [
        pltpu.VMEM((_B0, 128), jnp.int32),
        pltpu.VMEM((_B0, 128), jnp.int32),
        pltpu.VMEM((nbuf, 128, D), jnp.float32),
        pltpu.VMEM((16, D), jnp.float32),
        pltpu.VMEM_SHARED((_NPAD, D), jnp.float32),
        pltpu.SemaphoreType.DMA,
        pltpu.SemaphoreType.DMA,
    ]
    if with_deg:
        out_type.append(jax.ShapeDtypeStruct((2, _NPAD // 128, 128),
                                             jnp.float32))
        scratch += [
            pltpu.VMEM((1, 128), jnp.int32),
            pltpu.VMEM((1, 128), jnp.int32),
            pltpu.VMEM((128, D), jnp.float32),
            pltpu.VMEM_SHARED((_NPAD // 128, 128), jnp.float32),
            pltpu.SemaphoreType.DMA,
        ]

    @functools.partial(pl.kernel, out_type=out_type, mesh=mesh,
                       scratch_types=scratch)
    def segsum(*args):
        if with_deg:
            (feat, src2d, dst2d, eye, out, dout,
             sidx, didx, rows, zbuf, acc, semg, sems, dmrow, ddrow, orows,
             dacc, seme) = args
        else:
            (feat, src2d, dst2d, out,
             sidx, didx, rows, zbuf, acc, semg, sems) = args
        c = lax.axis_index("c")
        s = lax.axis_index("s")
        nb = jnp.where(c == 0, _B0, _B1)
        row0 = jnp.where(c == 0, s * _B0, 16 * _B0 + s * _B1)

        def zrow(r, carry):
            for j in range(D // 16):
                zbuf[r, pl.ds(j * 16, 16)] = jnp.zeros((16,), jnp.float32)
            return carry

        lax.fori_loop(0, 16, zrow, 0)
        for i in range(40):
            pltpu.sync_copy(zbuf, acc.at[pl.ds(s * _RPS + i * 16, 16)])
        if with_deg:
            @pl.when(s == 0)
            def _():
                for i in range(_NPAD // 128 // 16):
                    pltpu.sync_copy(zbuf, dacc.at[pl.ds(i * 16, 16)])

        plsc.subcore_barrier()

        pltpu.sync_copy(src2d.at[pl.ds(row0, _B0)], sidx)
        pltpu.sync_copy(dst2d.at[pl.ds(row0, _B0)], didx)

        if with_deg:
            buf = rows.at[0]
            pltpu.async_copy(feat.at[sidx.at[0]], buf, semg)

            def batch(j, carry):
                pltpu.make_async_copy(feat.at[sidx.at[j]], buf, semg).wait()
                pltpu.async_copy(buf, acc.at[didx.at[j]], sems, add=True)
                for k in range(8):
                    dv = didx[j, pl.ds(k * 16, 16)]
                    dmrow[0, pl.ds(k * 16, 16)] = lax.rem(dv, 128)
                    ddrow[0, pl.ds(k * 16, 16)] = lax.div(dv, 128)
                pltpu.async_copy(eye.at[dmrow.at[0]], orows, seme)
                pltpu.make_async_copy(buf, acc.at[didx.at[j]], sems).wait()

                @pl.when(j + 1 < nb)
                def _g():
                    pltpu.async_copy(feat.at[sidx.at[j + 1]], buf, semg)

                pltpu.make_async_copy(eye.at[dmrow.at[0]], orows,
                                      seme).wait()
                pltpu.sync_copy(orows, dacc.at[ddrow.at[0]], add=True)
                return carry

            lax.fori_loop(0, nb, batch, 0)
        else:
            pltpu.async_copy(feat.at[sidx.at[0]], rows.at[0], semg)

            def halfstep(j, b):
                pltpu.make_async_copy(feat.at[sidx.at[j]], rows.at[b],
                                      semg).wait()

                @pl.when(j >= 1)
                def _sw():
                    pltpu.make_async_copy(rows.at[1 - b],
                                          acc.at[didx.at[j - 1]], sems).wait()

                @pl.when(j + 1 < nb)
                def _gs():
                    pltpu.async_copy(feat.at[sidx.at[j + 1]], rows.at[1 - b],
                                     semg)

                pltpu.async_copy(rows.at[b], acc.at[didx.at[j]], sems,
                                 add=True)

            def pair(p, carry):
                halfstep(2 * p, 0)
                halfstep(2 * p + 1, 1)
                return carry

            lax.fori_loop(0, nb // 2, pair, 0)
            pltpu.make_async_copy(rows.at[1], acc.at[didx.at[nb - 1]],
                                  sems).wait()
        plsc.subcore_barrier()

        for i in range(40):
            r0 = s * _RPS + i * 16
            pltpu.sync_copy(acc.at[pl.ds(r0, 16)], zbuf)
            pltpu.sync_copy(zbuf, out.at[c].at[pl.ds(r0, 16)])
        if with_deg:
            @pl.when(s == 1)
            def _():
                pltpu.sync_copy(dacc, orows.at[pl.ds(0, _NPAD // 128)])
                pltpu.sync_copy(orows.at[pl.ds(0, _NPAD // 128)], dout.at[c])

    return segsum


_sc_segsum_deg = _make_sc_segsum(True)
_sc_segsum = _make_sc_segsum(False)


def kernel(x, edge_index, y, mask, W1l, b1l, W1r, W2l, b2l, W2r):
    npad = _NBR * 128 - E
    src2d = jnp.concatenate(
        [edge_index[0], jnp.zeros((npad,), jnp.int32)]).reshape(_NBR, 128)
    dst_flat = jnp.concatenate(
        [edge_index[1], jnp.full((npad,), N + 16, jnp.int32)])
    dst2d = dst_flat.reshape(_NBR, 128)
    eye = jnp.eye(128, dtype=jnp.float32)

    parts1, degp = _sc_segsum_deg(x, src2d, dst2d, eye)
    deg0 = degp[0].reshape(_NPAD, 1)
    deg1 = degp[1].reshape(_NPAD, 1)
    h, hn, onehot = _layer1(parts1[0], parts1[1], deg0, deg1, x, W1l, b1l,
                            W1r, y.reshape(N, 1))

    (parts2,) = _sc_segsum(h, src2d, dst2d)
    p_sim = _fused_sim(hn, mask, onehot)
    final = _layer2(parts2[0], parts2[1], deg0, deg1, h, W2l, b2l, W2r, p_sim)
    return (final, h)
